# baseline (speedup 1.0000x reference)
import functools

import jax
import jax.numpy as jnp
from jax import lax
from jax.experimental import pallas as pl
from jax.experimental.pallas import tpu as pltpu
from jax.experimental.pallas import tpu_sc as plsc

N = 10000
E = 320000
D = 128

NC = 2
NS = 16
NW = NC * NS

E_PER_TILE = E // NW
CHUNK = 80
NCHUNK = E_PER_TILE // CHUNK
ROWS_PER_TILE = N // NS
OUT_SLAB = 624
OUT_TAIL = N - NS * OUT_SLAB
LANES = 16
DGRP = D // LANES


def _mm_body(x_ref, w_ref, o_ref):
    o_ref[...] = jnp.dot(x_ref[...], w_ref[...],
                         preferred_element_type=jnp.float32)


def _matmul(x, W):
    mblk = 2000
    return pl.pallas_call(
        _mm_body,
        grid=(N // mblk,),
        in_specs=[
            pl.BlockSpec((mblk, D), lambda i: (i, 0)),
            pl.BlockSpec((D, D), lambda i: (0, 0)),
        ],
        out_specs=pl.BlockSpec((mblk, D), lambda i: (i, 0)),
        out_shape=jax.ShapeDtypeStruct((N, D), jnp.float32),
    )(x, W)


NB = 4


def _spmm_body(src_hbm, dst_hbm, vals_hbm, support_hbm, out_hbm,
               sbuf0, sbuf1, sbuf2, sbuf3,
               dbuf0, dbuf1, dbuf2, dbuf3,
               vbuf0, vbuf1, vbuf2, vbuf3,
               gbuf0, gbuf1, gbuf2, gbuf3, acc,
               esem0, esem1, esem2, esem3,
               dsem0, dsem1, dsem2, dsem3,
               gsem0, gsem1, gsem2, gsem3,
               ssem0, ssem1, ssem2, ssem3):
    c = lax.axis_index("c")
    s = lax.axis_index("s")
    wid = s * NC + c
    ebase = wid * E_PER_TILE

    sbufs = (sbuf0, sbuf1, sbuf2, sbuf3)
    dbufs = (dbuf0, dbuf1, dbuf2, dbuf3)
    vbufs = (vbuf0, vbuf1, vbuf2, vbuf3)
    gbufs = (gbuf0, gbuf1, gbuf2, gbuf3)
    esems = (esem0, esem1, esem2, esem3)
    dsems = (dsem0, dsem1, dsem2, dsem3)
    gsems = (gsem0, gsem1, gsem2, gsem3)
    ssems = (ssem0, ssem1, ssem2, ssem3)

    def estart(j, b):
        pltpu.async_copy(src_hbm.at[pl.ds(ebase + j * CHUNK, CHUNK)],
                         sbufs[b], esems[b])
        pltpu.async_copy(vals_hbm.at[pl.ds(ebase + j * CHUNK, CHUNK)],
                         vbufs[b], esems[b])

    def ewait(b):
        pltpu.make_async_copy(src_hbm.at[pl.ds(0, CHUNK)],
                              sbufs[b], esems[b]).wait()
        pltpu.make_async_copy(vals_hbm.at[pl.ds(0, CHUNK)],
                              vbufs[b], esems[b]).wait()

    def dstart(j, b):
        pltpu.async_copy(dst_hbm.at[pl.ds(ebase + j * CHUNK, CHUNK)],
                         dbufs[b], dsems[b])

    def dwait(b):
        pltpu.make_async_copy(dst_hbm.at[pl.ds(0, CHUNK)],
                              dbufs[b], dsems[b]).wait()

    def gstart(b):
        pltpu.async_copy(support_hbm.at[sbufs[b]], gbufs[b], gsems[b])

    def gwait(b):
        pltpu.make_async_copy(support_hbm.at[sbufs[b]],
                              gbufs[b], gsems[b]).wait()

    def sstart(b):
        pltpu.async_copy(gbufs[b], acc.at[dbufs[b]], ssems[b], add=True)

    def swait(b):
        pltpu.make_async_copy(gbufs[b], acc.at[dbufs[b]], ssems[b]).wait()

    def scale(b):
        gb, vb = gbufs[b], vbufs[b]

        def sgrp(g, inner):
            vals16 = vb[pl.ds(g * LANES, LANES)]
            row0 = g * LANES
            for r in range(LANES):
                val = jnp.broadcast_to(vals16[r], (LANES,))
                row = row0 + r
                for k in range(DGRP):
                    sl = pl.ds(k * LANES, LANES)
                    gb[row, sl] = gb[row, sl] * val
            return inner

        lax.fori_loop(0, CHUNK // LANES, sgrp, 0)

    def run(j, b, with_estart, with_gstart, with_swait, with_dstart):
        if with_estart:
            estart(j + 3, (b + 3) % NB)
        if with_gstart:
            ewait((b + 1) % NB)
            gstart((b + 1) % NB)
        gwait(b)
        if with_swait:
            swait((b + 2) % NB)
        if with_dstart:
            dstart(j + 2, (b + 2) % NB)
        scale(b)
        dwait(b)
        sstart(b)

    estart(0, 0)
    estart(1, 1)
    estart(2, 2)
    dstart(0, 0)
    dstart(1, 1)

    zero = jnp.zeros((LANES,), jnp.float32)

    def zrow(i, carry):
        for k in range(DGRP):
            gbuf3[i, pl.ds(k * LANES, LANES)] = zero
        return carry

    lax.fori_loop(0, CHUNK, zrow, 0)
    rbase = s * ROWS_PER_TILE
    for q in range(ROWS_PER_TILE // CHUNK):
        pltpu.sync_copy(gbuf3, acc.at[pl.ds(rbase + q * CHUNK, CHUNK)])
    rem = ROWS_PER_TILE % CHUNK
    if rem:
        pltpu.sync_copy(gbuf3.at[pl.ds(0, rem)],
                        acc.at[pl.ds(rbase + (ROWS_PER_TILE // CHUNK) * CHUNK,
                                     rem)])

    ewait(0)
    gstart(0)
    plsc.subcore_barrier()

    run(0, 0, True, True, False, True)
    run(1, 1, True, True, False, True)
    run(2, 2, True, True, True, True)
    run(3, 3, True, True, True, True)

    def steady(j0, carry):
        for u in range(NB):
            run(4 + NB * j0 + u, u, True, True, True, True)
        return carry

    lax.fori_loop(0, (NCHUNK - 9) // NB, steady, 0)

    run(NCHUNK - 5, 0, True, True, True, True)
    run(NCHUNK - 4, 1, True, True, True, True)
    run(NCHUNK - 3, 2, False, True, True, True)
    run(NCHUNK - 2, 3, False, True, True, False)
    run(NCHUNK - 1, 0, False, False, True, False)
    swait(3)
    swait(0)
    plsc.subcore_barrier()

    obase = s * OUT_SLAB
    pltpu.sync_copy(acc.at[pl.ds(obase, OUT_SLAB)],
                    out_hbm.at[c, pl.ds(obase, OUT_SLAB)])

    @pl.when(s == NS - 1)
    def _tail():
        pltpu.sync_copy(acc.at[pl.ds(NS * OUT_SLAB, OUT_TAIL)],
                        out_hbm.at[c, pl.ds(NS * OUT_SLAB, OUT_TAIL)])


_spmm = functools.partial(
    pl.kernel,
    out_type=jax.ShapeDtypeStruct((NC, N, D), jnp.float32),
    mesh=plsc.VectorSubcoreMesh(core_axis_name="c", subcore_axis_name="s"),
    scratch_types=(
        [pltpu.VMEM((CHUNK,), jnp.int32) for _ in range(NB)]
        + [pltpu.VMEM((CHUNK,), jnp.int32) for _ in range(NB)]
        + [pltpu.VMEM((CHUNK,), jnp.float32) for _ in range(NB)]
        + [pltpu.VMEM((CHUNK, D), jnp.float32) for _ in range(NB)]
        + [pltpu.VMEM_SHARED((N, D), jnp.float32)]
        + [pltpu.SemaphoreType.DMA for _ in range(4 * NB)]
    ),
)(_spmm_body)


def _add_body(a_ref, b_ref, o_ref):
    o_ref[...] = a_ref[...] + b_ref[...]


def _combine(partials):
    mblk = 2000
    return pl.pallas_call(
        _add_body,
        grid=(N // mblk,),
        in_specs=[
            pl.BlockSpec((1, mblk, D), lambda i: (0, i, 0)),
            pl.BlockSpec((1, mblk, D), lambda i: (1, i, 0)),
        ],
        out_specs=pl.BlockSpec((1, mblk, D), lambda i: (0, i, 0)),
        out_shape=jax.ShapeDtypeStruct((1, N, D), jnp.float32),
    )(partials, partials)[0]


@jax.jit
def kernel(x, edge_index, adj_vals, W):
    support = _matmul(x, W)
    partials = _spmm(edge_index[1], edge_index[0], adj_vals, support)
    return _combine(partials)


# speedup vs baseline: 1.0969x; 1.0969x over previous
import functools

import jax
import jax.numpy as jnp
from jax import lax
from jax.experimental import pallas as pl
from jax.experimental.pallas import tpu as pltpu
from jax.experimental.pallas import tpu_sc as plsc

N = 10000
E = 320000
D = 128

NC = 2
NS = 16
NW = NC * NS

E_PER_TILE = E // NW
CHUNK = 80
NCHUNK = E_PER_TILE // CHUNK
ROWS_PER_TILE = N // NS
OUT_SLAB = 624
OUT_TAIL = N - NS * OUT_SLAB
LANES = 16
DGRP = D // LANES


def _mm_body(x_ref, w_ref, e_ref, o_ref, dst_ref, src_ref):
    o_ref[...] = jnp.dot(x_ref[...], w_ref[...],
                         preferred_element_type=jnp.float32)
    dst_ref[...] = e_ref[0, :]
    src_ref[...] = e_ref[1, :]


def _matmul_split(x, W, edge_index):
    mblk = 2000
    return pl.pallas_call(
        _mm_body,
        grid=(N // mblk,),
        in_specs=[
            pl.BlockSpec((mblk, D), lambda i: (i, 0)),
            pl.BlockSpec((D, D), lambda i: (0, 0)),
            pl.BlockSpec((2, E), lambda i: (0, 0)),
        ],
        out_specs=[
            pl.BlockSpec((mblk, D), lambda i: (i, 0)),
            pl.BlockSpec((E,), lambda i: (0,)),
            pl.BlockSpec((E,), lambda i: (0,)),
        ],
        out_shape=[
            jax.ShapeDtypeStruct((N, D), jnp.float32),
            jax.ShapeDtypeStruct((E,), jnp.int32),
            jax.ShapeDtypeStruct((E,), jnp.int32),
        ],
    )(x, W, edge_index)


NB = 4


def _spmm_body(src_hbm, dst_hbm, vals_hbm, support_hbm, out_hbm,
               sbuf0, sbuf1, sbuf2, sbuf3,
               dbuf0, dbuf1, dbuf2, dbuf3,
               vbuf0, vbuf1, vbuf2, vbuf3,
               gbuf0, gbuf1, gbuf2, gbuf3, acc,
               esem0, esem1, esem2, esem3,
               dsem0, dsem1, dsem2, dsem3,
               gsem0, gsem1, gsem2, gsem3,
               ssem0, ssem1, ssem2, ssem3):
    c = lax.axis_index("c")
    s = lax.axis_index("s")
    wid = s * NC + c
    ebase = wid * E_PER_TILE

    sbufs = (sbuf0, sbuf1, sbuf2, sbuf3)
    dbufs = (dbuf0, dbuf1, dbuf2, dbuf3)
    vbufs = (vbuf0, vbuf1, vbuf2, vbuf3)
    gbufs = (gbuf0, gbuf1, gbuf2, gbuf3)
    esems = (esem0, esem1, esem2, esem3)
    dsems = (dsem0, dsem1, dsem2, dsem3)
    gsems = (gsem0, gsem1, gsem2, gsem3)
    ssems = (ssem0, ssem1, ssem2, ssem3)

    def estart(j, b):
        pltpu.async_copy(src_hbm.at[pl.ds(ebase + j * CHUNK, CHUNK)],
                         sbufs[b], esems[b])
        pltpu.async_copy(vals_hbm.at[pl.ds(ebase + j * CHUNK, CHUNK)],
                         vbufs[b], esems[b])

    def ewait(b):
        pltpu.make_async_copy(src_hbm.at[pl.ds(0, CHUNK)],
                              sbufs[b], esems[b]).wait()
        pltpu.make_async_copy(vals_hbm.at[pl.ds(0, CHUNK)],
                              vbufs[b], esems[b]).wait()

    def dstart(j, b):
        pltpu.async_copy(dst_hbm.at[pl.ds(ebase + j * CHUNK, CHUNK)],
                         dbufs[b], dsems[b])

    def dwait(b):
        pltpu.make_async_copy(dst_hbm.at[pl.ds(0, CHUNK)],
                              dbufs[b], dsems[b]).wait()

    def gstart(b):
        pltpu.async_copy(support_hbm.at[sbufs[b]], gbufs[b], gsems[b])

    def gwait(b):
        pltpu.make_async_copy(support_hbm.at[sbufs[b]],
                              gbufs[b], gsems[b]).wait()

    def sstart(b):
        pltpu.async_copy(gbufs[b], acc.at[dbufs[b]], ssems[b], add=True)

    def swait(b):
        pltpu.make_async_copy(gbufs[b], acc.at[dbufs[b]], ssems[b]).wait()

    def scale(b):
        gb, vb = gbufs[b], vbufs[b]

        def sgrp(g, inner):
            vals16 = vb[pl.ds(g * LANES, LANES)]
            row0 = g * LANES
            for r in range(LANES):
                val = jnp.broadcast_to(vals16[r], (LANES,))
                row = row0 + r
                for k in range(DGRP):
                    sl = pl.ds(k * LANES, LANES)
                    gb[row, sl] = gb[row, sl] * val
            return inner

        lax.fori_loop(0, CHUNK // LANES, sgrp, 0)

    def run(j, b, with_estart, with_gstart, with_swait, with_dstart):
        if with_estart:
            estart(j + 3, (b + 3) % NB)
        if with_gstart:
            ewait((b + 1) % NB)
            gstart((b + 1) % NB)
        gwait(b)
        if with_swait:
            swait((b + 2) % NB)
        if with_dstart:
            dstart(j + 2, (b + 2) % NB)
        scale(b)
        dwait(b)
        sstart(b)

    estart(0, 0)
    estart(1, 1)
    estart(2, 2)
    dstart(0, 0)
    dstart(1, 1)

    zero = jnp.zeros((LANES,), jnp.float32)

    def zrow(i, carry):
        for k in range(DGRP):
            gbuf3[i, pl.ds(k * LANES, LANES)] = zero
        return carry

    lax.fori_loop(0, CHUNK, zrow, 0)
    rbase = s * ROWS_PER_TILE
    for q in range(ROWS_PER_TILE // CHUNK):
        pltpu.sync_copy(gbuf3, acc.at[pl.ds(rbase + q * CHUNK, CHUNK)])
    rem = ROWS_PER_TILE % CHUNK
    if rem:
        pltpu.sync_copy(gbuf3.at[pl.ds(0, rem)],
                        acc.at[pl.ds(rbase + (ROWS_PER_TILE // CHUNK) * CHUNK,
                                     rem)])

    ewait(0)
    gstart(0)
    plsc.subcore_barrier()

    run(0, 0, True, True, False, True)
    run(1, 1, True, True, False, True)
    run(2, 2, True, True, True, True)
    run(3, 3, True, True, True, True)

    def steady(j0, carry):
        for u in range(NB):
            run(4 + NB * j0 + u, u, True, True, True, True)
        return carry

    lax.fori_loop(0, (NCHUNK - 9) // NB, steady, 0)

    run(NCHUNK - 5, 0, True, True, True, True)
    run(NCHUNK - 4, 1, True, True, True, True)
    run(NCHUNK - 3, 2, False, True, True, True)
    run(NCHUNK - 2, 3, False, True, True, False)
    run(NCHUNK - 1, 0, False, False, True, False)
    swait(3)
    swait(0)
    plsc.subcore_barrier()

    obase = s * OUT_SLAB
    pltpu.sync_copy(acc.at[pl.ds(obase, OUT_SLAB)],
                    out_hbm.at[c, pl.ds(obase, OUT_SLAB)])

    @pl.when(s == NS - 1)
    def _tail():
        pltpu.sync_copy(acc.at[pl.ds(NS * OUT_SLAB, OUT_TAIL)],
                        out_hbm.at[c, pl.ds(NS * OUT_SLAB, OUT_TAIL)])


_spmm = functools.partial(
    pl.kernel,
    out_type=jax.ShapeDtypeStruct((NC, N, D), jnp.float32),
    mesh=plsc.VectorSubcoreMesh(core_axis_name="c", subcore_axis_name="s"),
    scratch_types=(
        [pltpu.VMEM((CHUNK,), jnp.int32) for _ in range(NB)]
        + [pltpu.VMEM((CHUNK,), jnp.int32) for _ in range(NB)]
        + [pltpu.VMEM((CHUNK,), jnp.float32) for _ in range(NB)]
        + [pltpu.VMEM((CHUNK, D), jnp.float32) for _ in range(NB)]
        + [pltpu.VMEM_SHARED((N, D), jnp.float32)]
        + [pltpu.SemaphoreType.DMA for _ in range(4 * NB)]
    ),
)(_spmm_body)


def _add_body(a_ref, b_ref, o_ref):
    o_ref[...] = a_ref[...] + b_ref[...]


def _combine(partials):
    mblk = 2000
    return pl.pallas_call(
        _add_body,
        grid=(N // mblk,),
        in_specs=[
            pl.BlockSpec((1, mblk, D), lambda i: (0, i, 0)),
            pl.BlockSpec((1, mblk, D), lambda i: (1, i, 0)),
        ],
        out_specs=pl.BlockSpec((1, mblk, D), lambda i: (0, i, 0)),
        out_shape=jax.ShapeDtypeStruct((1, N, D), jnp.float32),
    )(partials, partials)[0]


@jax.jit
def kernel(x, edge_index, adj_vals, W):
    support, dst, src = _matmul_split(x, W, edge_index)
    partials = _spmm(src, dst, adj_vals, support)
    return _combine(partials)


# speedup vs baseline: 1.1208x; 1.0218x over previous
import functools

import jax
import jax.numpy as jnp
from jax import lax
from jax.experimental import pallas as pl
from jax.experimental.pallas import tpu as pltpu
from jax.experimental.pallas import tpu_sc as plsc

N = 10000
E = 320000
D = 128

NC = 2
NS = 16
NW = NC * NS

E_PER_TILE = E // NW
CHUNK = 80
NCHUNK = E_PER_TILE // CHUNK
ROWS_PER_TILE = N // NS
OUT_SLAB = 624
OUT_TAIL = N - NS * OUT_SLAB
LANES = 16
DGRP = D // LANES


def _mm_body(x_ref, w_ref, e_ref, o_ref, dst_ref, src_ref):
    o_ref[...] = jnp.dot(x_ref[...], w_ref[...],
                         preferred_element_type=jnp.float32)
    dst_ref[...] = e_ref[0, :]
    src_ref[...] = e_ref[1, :]


def _matmul_split(x, W, edge_index):
    mblk = 2000
    return pl.pallas_call(
        _mm_body,
        grid=(N // mblk,),
        in_specs=[
            pl.BlockSpec((mblk, D), lambda i: (i, 0)),
            pl.BlockSpec((D, D), lambda i: (0, 0)),
            pl.BlockSpec((2, E), lambda i: (0, 0)),
        ],
        out_specs=[
            pl.BlockSpec((mblk, D), lambda i: (i, 0)),
            pl.BlockSpec((E,), lambda i: (0,)),
            pl.BlockSpec((E,), lambda i: (0,)),
        ],
        out_shape=[
            jax.ShapeDtypeStruct((N, D), jnp.float32),
            jax.ShapeDtypeStruct((E,), jnp.int32),
            jax.ShapeDtypeStruct((E,), jnp.int32),
        ],
    )(x, W, edge_index)


NB = 4


def _spmm_body(src_hbm, dst_hbm, vals_hbm, support_hbm, out_hbm,
               sbuf0, sbuf1, sbuf2, sbuf3,
               dbuf0, dbuf1, dbuf2, dbuf3,
               vbuf0, vbuf1, vbuf2, vbuf3,
               gbuf0, gbuf1, gbuf2, gbuf3, acc,
               esem0, esem1, esem2, esem3,
               dsem0, dsem1, dsem2, dsem3,
               gsem0, gsem1, gsem2, gsem3,
               ssem0, ssem1, ssem2, ssem3):
    c = lax.axis_index("c")
    s = lax.axis_index("s")
    wid = s * NC + c
    ebase = wid * E_PER_TILE

    sbufs = (sbuf0, sbuf1, sbuf2, sbuf3)
    dbufs = (dbuf0, dbuf1, dbuf2, dbuf3)
    vbufs = (vbuf0, vbuf1, vbuf2, vbuf3)
    gbufs = (gbuf0, gbuf1, gbuf2, gbuf3)
    esems = (esem0, esem1, esem2, esem3)
    dsems = (dsem0, dsem1, dsem2, dsem3)
    gsems = (gsem0, gsem1, gsem2, gsem3)
    ssems = (ssem0, ssem1, ssem2, ssem3)

    def estart(j, b):
        pltpu.async_copy(src_hbm.at[pl.ds(ebase + j * CHUNK, CHUNK)],
                         sbufs[b], esems[b])
        pltpu.async_copy(vals_hbm.at[pl.ds(ebase + j * CHUNK, CHUNK)],
                         vbufs[b], esems[b])

    def ewait(b):
        pltpu.make_async_copy(src_hbm.at[pl.ds(0, CHUNK)],
                              sbufs[b], esems[b]).wait()
        pltpu.make_async_copy(vals_hbm.at[pl.ds(0, CHUNK)],
                              vbufs[b], esems[b]).wait()

    def dstart(j, b):
        pltpu.async_copy(dst_hbm.at[pl.ds(ebase + j * CHUNK, CHUNK)],
                         dbufs[b], dsems[b])

    def dwait(b):
        pltpu.make_async_copy(dst_hbm.at[pl.ds(0, CHUNK)],
                              dbufs[b], dsems[b]).wait()

    def gstart(b):
        pltpu.async_copy(support_hbm.at[sbufs[b]], gbufs[b], gsems[b])

    def gwait(b):
        pltpu.make_async_copy(support_hbm.at[sbufs[b]],
                              gbufs[b], gsems[b]).wait()

    def sstart(b):
        pltpu.async_copy(gbufs[b], acc.at[dbufs[b]], ssems[b], add=True)

    def swait(b):
        pltpu.make_async_copy(gbufs[b], acc.at[dbufs[b]], ssems[b]).wait()

    def scale(b):
        gb, vb = gbufs[b], vbufs[b]

        def sgrp(g, inner):
            vals16 = vb[pl.ds(g * LANES, LANES)]
            row0 = g * LANES
            for r in range(LANES):
                val = jnp.broadcast_to(vals16[r], (LANES,))
                row = row0 + r
                for k in range(DGRP):
                    sl = pl.ds(k * LANES, LANES)
                    gb[row, sl] = gb[row, sl] * val
            return inner

        lax.fori_loop(0, CHUNK // LANES, sgrp, 0)

    def run(j, b, with_estart, with_gstart, with_swait, with_dstart):
        if with_estart:
            estart(j + 3, (b + 3) % NB)
        gwait(b)
        if with_swait:
            swait((b + 2) % NB)
        if with_gstart:
            ewait((b + 2) % NB)
            gstart((b + 2) % NB)
        if with_dstart:
            dstart(j + 2, (b + 2) % NB)
        scale(b)
        dwait(b)
        sstart(b)

    estart(0, 0)
    estart(1, 1)
    estart(2, 2)
    dstart(0, 0)
    dstart(1, 1)

    zero = jnp.zeros((LANES,), jnp.float32)

    def zrow(i, carry):
        for k in range(DGRP):
            gbuf3[i, pl.ds(k * LANES, LANES)] = zero
        return carry

    lax.fori_loop(0, CHUNK, zrow, 0)
    rbase = s * ROWS_PER_TILE
    for q in range(ROWS_PER_TILE // CHUNK):
        pltpu.sync_copy(gbuf3, acc.at[pl.ds(rbase + q * CHUNK, CHUNK)])
    rem = ROWS_PER_TILE % CHUNK
    if rem:
        pltpu.sync_copy(gbuf3.at[pl.ds(0, rem)],
                        acc.at[pl.ds(rbase + (ROWS_PER_TILE // CHUNK) * CHUNK,
                                     rem)])

    ewait(0)
    gstart(0)
    ewait(1)
    gstart(1)
    plsc.subcore_barrier()

    run(0, 0, True, True, False, True)
    run(1, 1, True, True, False, True)
    run(2, 2, True, True, True, True)
    run(3, 3, True, True, True, True)

    def steady(j0, carry):
        for u in range(NB):
            run(4 + NB * j0 + u, u, True, True, True, True)
        return carry

    lax.fori_loop(0, (NCHUNK - 9) // NB, steady, 0)

    run(NCHUNK - 5, 0, True, True, True, True)
    run(NCHUNK - 4, 1, True, True, True, True)
    run(NCHUNK - 3, 2, False, True, True, True)
    run(NCHUNK - 2, 3, False, False, True, False)
    run(NCHUNK - 1, 0, False, False, True, False)
    swait(3)
    swait(0)
    plsc.subcore_barrier()

    obase = s * OUT_SLAB
    pltpu.sync_copy(acc.at[pl.ds(obase, OUT_SLAB)],
                    out_hbm.at[c, pl.ds(obase, OUT_SLAB)])

    @pl.when(s == NS - 1)
    def _tail():
        pltpu.sync_copy(acc.at[pl.ds(NS * OUT_SLAB, OUT_TAIL)],
                        out_hbm.at[c, pl.ds(NS * OUT_SLAB, OUT_TAIL)])


_spmm = functools.partial(
    pl.kernel,
    out_type=jax.ShapeDtypeStruct((NC, N, D), jnp.float32),
    mesh=plsc.VectorSubcoreMesh(core_axis_name="c", subcore_axis_name="s"),
    scratch_types=(
        [pltpu.VMEM((CHUNK,), jnp.int32) for _ in range(NB)]
        + [pltpu.VMEM((CHUNK,), jnp.int32) for _ in range(NB)]
        + [pltpu.VMEM((CHUNK,), jnp.float32) for _ in range(NB)]
        + [pltpu.VMEM((CHUNK, D), jnp.float32) for _ in range(NB)]
        + [pltpu.VMEM_SHARED((N, D), jnp.float32)]
        + [pltpu.SemaphoreType.DMA for _ in range(4 * NB)]
    ),
)(_spmm_body)


def _add_body(a_ref, b_ref, o_ref):
    o_ref[...] = a_ref[...] + b_ref[...]


def _combine(partials):
    mblk = 2000
    return pl.pallas_call(
        _add_body,
        grid=(N // mblk,),
        in_specs=[
            pl.BlockSpec((1, mblk, D), lambda i: (0, i, 0)),
            pl.BlockSpec((1, mblk, D), lambda i: (1, i, 0)),
        ],
        out_specs=pl.BlockSpec((1, mblk, D), lambda i: (0, i, 0)),
        out_shape=jax.ShapeDtypeStruct((1, N, D), jnp.float32),
    )(partials, partials)[0]


@jax.jit
def kernel(x, edge_index, adj_vals, W):
    support, dst, src = _matmul_split(x, W, edge_index)
    partials = _spmm(src, dst, adj_vals, support)
    return _combine(partials)
